# Initial kernel scaffold; baseline (speedup 1.0000x reference)
import jax
import jax.numpy as jnp
from jax.experimental import pallas as pl


def kernel(h, edge_index):
    raise NotImplementedError("write your pallas kernel here")


# speedup vs baseline: 3.7040x; 3.7040x over previous
import functools

import jax
import jax.numpy as jnp
from jax import lax
from jax.experimental import pallas as pl
from jax.experimental.pallas import tpu as pltpu
from jax.experimental.pallas import tpu_sc as plsc

def _lane_perm(x, idx):
    dnums = lax.GatherDimensionNumbers(
        offset_dims=(), collapsed_slice_dims=(0,), start_index_map=(0,))
    return lax.gather(x, idx[:, None], dnums, slice_sizes=(1,),
                      mode=lax.GatherScatterMode.PROMISE_IN_BOUNDS)


N_NODES = 10000
N_EDGES = 320000
D_FEAT = 128

NUM_WORKERS = 32
EDGES_PER_WORKER = N_EDGES // NUM_WORKERS
CHUNK = 400
NUM_CHUNKS = EDGES_PER_WORKER // CHUNK


@functools.partial(
    pl.kernel,
    mesh=plsc.VectorSubcoreMesh(core_axis_name="c", subcore_axis_name="s"),
    out_type=jax.ShapeDtypeStruct((N_EDGES,), jnp.float32),
    scratch_types=[
        pltpu.VMEM((CHUNK,), jnp.int32),
        pltpu.VMEM((CHUNK,), jnp.int32),
        pltpu.VMEM((CHUNK, D_FEAT), jnp.float32),
        pltpu.VMEM((CHUNK, D_FEAT), jnp.float32),
        pltpu.VMEM((CHUNK,), jnp.float32),
        pltpu.SemaphoreType.DMA,
    ],
)
def _dot_scores(h_hbm, eidx_hbm, out_hbm, su_v, sv_v, hu_v, hv_v, out_v, sem):
    wid = lax.axis_index("s") * 2 + lax.axis_index("c")
    wbase = wid * EDGES_PER_WORKER
    lane = lax.iota(jnp.int32, 16)

    def chunk_body(c, carry):
        base = wbase + c * CHUNK
        pltpu.sync_copy(eidx_hbm.at[pl.ds(base, CHUNK)], su_v)
        pltpu.sync_copy(eidx_hbm.at[pl.ds(N_EDGES + base, CHUNK)], sv_v)
        cp_u = pltpu.async_copy(h_hbm.at[su_v], hu_v, sem)
        cp_v = pltpu.async_copy(h_hbm.at[sv_v], hv_v, sem)
        cp_u.wait()
        cp_v.wait()

        def group_body(g, carry2):
            vec = jnp.zeros((16,), jnp.float32)
            for i in range(16):
                r = g * 16 + i
                acc = hu_v[r, pl.ds(0, 16)] * hv_v[r, pl.ds(0, 16)]
                for j in range(1, D_FEAT // 16):
                    acc = acc + (hu_v[r, pl.ds(j * 16, 16)]
                                 * hv_v[r, pl.ds(j * 16, 16)])
                for k in (8, 4, 2, 1):
                    acc = acc + _lane_perm(acc, lane ^ k)
                vec = jnp.where(lane == i, acc, vec)
            out_v[pl.ds(g * 16, 16)] = vec
            return carry2

        lax.fori_loop(0, CHUNK // 16, group_body, 0)
        pltpu.sync_copy(out_v, out_hbm.at[pl.ds(base, CHUNK)])
        return carry

    lax.fori_loop(0, NUM_CHUNKS, chunk_body, 0)


def kernel(h, edge_index):
    eidx = edge_index.astype(jnp.int32).reshape(-1)
    score = _dot_scores(h, eidx)
    return score.reshape(N_EDGES, 1)


# speedup vs baseline: 7.2368x; 1.9538x over previous
import functools

import jax
import jax.numpy as jnp
from jax import lax
from jax.experimental import pallas as pl
from jax.experimental.pallas import tpu as pltpu
from jax.experimental.pallas import tpu_sc as plsc

N_NODES = 10000
N_EDGES = 320000
D_FEAT = 128
D_WORDS = D_FEAT // 2

NUM_SC = 2
NUM_TILES = 16
WORDS_PER_TILE = D_WORDS // NUM_TILES
EDGES_PER_SC = N_EDGES // NUM_SC
CHUNK = 2048
SLICE = CHUNK // NUM_TILES
NUM_CHUNKS = 80
NUM_ROUNDS = NUM_CHUNKS // 2
LAST_START = EDGES_PER_SC - CHUNK


def _bc_f32(x):
    return lax.bitcast_convert_type(x, jnp.float32)


@functools.partial(
    pl.kernel,
    mesh=plsc.VectorSubcoreMesh(core_axis_name="c", subcore_axis_name="s"),
    out_type=jax.ShapeDtypeStruct((N_EDGES,), jnp.float32),
    scratch_types=[
        pltpu.VMEM((WORDS_PER_TILE, N_NODES), jnp.int32),
        pltpu.VMEM((CHUNK,), jnp.int32),
        pltpu.VMEM((CHUNK,), jnp.int32),
        pltpu.VMEM((CHUNK,), jnp.int32),
        pltpu.VMEM((CHUNK,), jnp.int32),
        pltpu.VMEM((CHUNK,), jnp.float32),
        pltpu.VMEM((NUM_TILES, SLICE), jnp.float32),
        pltpu.VMEM((SLICE,), jnp.float32),
        pltpu.VMEM((SLICE,), jnp.float32),
        pltpu.SemaphoreType.DMA,
        pltpu.SemaphoreType.DMA,
        pltpu.SemaphoreType.DMA,
        pltpu.SemaphoreType.DMA,
        pltpu.VMEM_SHARED((2, NUM_TILES, CHUNK), jnp.float32),
    ],
    compiler_params=pltpu.CompilerParams(use_tc_tiling_on_sc=False,
                                         needs_layout_passes=False),
)
def _dot_scores(hp_hbm, eidx_hbm, out_hbm,
                tab, su0, sv0, su1, sv1, pbuf, red, fb0, fb1,
                semi0, semi1, semw0, semw1, part_sp):
    sc = lax.axis_index("c")
    t = lax.axis_index("s")
    sc_base = sc * EDGES_PER_SC

    pltpu.sync_copy(hp_hbm.at[t], tab)

    def chunk_start(c):
        return jnp.minimum(c * CHUNK, LAST_START)

    def start_idx_fetch(c, su, sv, semi):
        off = sc_base + chunk_start(c)
        pltpu.async_copy(eidx_hbm.at[pl.ds(off, CHUNK)], su, semi)
        pltpu.async_copy(eidx_hbm.at[pl.ds(N_EDGES + off, CHUNK)], sv, semi)

    def wait_idx_fetch(su, sv, semi):
        pltpu.make_async_copy(eidx_hbm.at[pl.ds(0, CHUNK)], su, semi).wait()
        pltpu.make_async_copy(eidx_hbm.at[pl.ds(0, CHUNK)], sv, semi).wait()

    def wait_writeback(fb, semw):
        pltpu.make_async_copy(fb, out_hbm.at[pl.ds(0, SLICE)], semw).wait()

    def compute_partials(su, sv):
        def group_body(g, carry):
            base = g * 16
            idxu = su[pl.ds(base, 16)]
            idxv = sv[pl.ds(base, 16)]
            acc = None
            for w in range(WORDS_PER_TILE):
                wsp = jnp.full((16,), w, jnp.int32)
                gu = plsc.load_gather(tab, [wsp, idxu])
                gv = plsc.load_gather(tab, [wsp, idxv])
                prod = (_bc_f32(gu << 16) * _bc_f32(gv << 16)
                        + _bc_f32(gu) * _bc_f32(gv))
                acc = prod if acc is None else acc + prod
            pbuf[pl.ds(base, 16)] = acc
            return carry

        lax.fori_loop(0, CHUNK // 16, group_body, 0)

    def reduce_and_write(c, fb, semw):
        for b in range(SLICE // 16):
            acc = red[0, pl.ds(b * 16, 16)]
            for j in range(1, NUM_TILES):
                acc = acc + red[j, pl.ds(b * 16, 16)]
            fb[pl.ds(b * 16, 16)] = acc
        off = sc_base + chunk_start(c) + t * SLICE
        pltpu.async_copy(fb, out_hbm.at[pl.ds(off, SLICE)], semw)

    start_idx_fetch(0, su0, sv0, semi0)
    start_idx_fetch(1, su1, sv1, semi1)

    rings = ((su0, sv0, semi0, fb0, semw0, 0),
             (su1, sv1, semi1, fb1, semw1, 1))

    def round_body(i, carry):
        for (su, sv, semi, fb, semw, k) in rings:
            c = 2 * i + k
            wait_idx_fetch(su, sv, semi)
            compute_partials(su, sv)
            pltpu.sync_copy(pbuf, part_sp.at[k, t])

            @pl.when(c + 2 < NUM_CHUNKS)
            def _():
                start_idx_fetch(c + 2, su, sv, semi)

            plsc.subcore_barrier()
            pltpu.sync_copy(part_sp.at[k, :, pl.ds(t * SLICE, SLICE)], red)

            @pl.when(i > 0)
            def _():
                wait_writeback(fb, semw)

            reduce_and_write(c, fb, semw)
        return carry

    lax.fori_loop(0, NUM_ROUNDS, round_body, 0)
    wait_writeback(fb0, semw0)
    wait_writeback(fb1, semw1)


def kernel(h, edge_index):
    eidx = edge_index.astype(jnp.int32).reshape(-1)
    h_packed = lax.bitcast_convert_type(
        h.astype(jnp.bfloat16).reshape(N_NODES, D_WORDS, 2), jnp.int32)
    hp = h_packed.T.reshape(NUM_TILES, WORDS_PER_TILE, N_NODES)
    score = _dot_scores(hp, eidx)
    return score.reshape(N_EDGES, 1)


# speedup vs baseline: 12.5052x; 1.7280x over previous
import functools

import jax
import jax.numpy as jnp
from jax import lax
from jax.experimental import pallas as pl
from jax.experimental.pallas import tpu as pltpu
from jax.experimental.pallas import tpu_sc as plsc


def _lane_perm(x, idx):
    dnums = lax.GatherDimensionNumbers(
        offset_dims=(), collapsed_slice_dims=(0,), start_index_map=(0,))
    return lax.gather(x, idx[:, None], dnums, slice_sizes=(1,),
                      mode=lax.GatherScatterMode.PROMISE_IN_BOUNDS)


N_NODES = 10000
N_EDGES = 320000
D_FEAT = 128
D_WORDS = D_FEAT // 2

NUM_WORKERS = 32
EDGES_PER_WORKER = N_EDGES // NUM_WORKERS
CHUNK = 256
RING = 2
NUM_CHUNKS = -(-EDGES_PER_WORKER // CHUNK)
NUM_ROUNDS = NUM_CHUNKS // RING
LAST_START = EDGES_PER_WORKER - CHUNK


@functools.partial(
    pl.kernel,
    mesh=plsc.VectorSubcoreMesh(core_axis_name="c", subcore_axis_name="s"),
    out_type=jax.ShapeDtypeStruct((N_EDGES,), jnp.float32),
    scratch_types=[
        pltpu.VMEM((EDGES_PER_WORKER,), jnp.int32),
        pltpu.VMEM((EDGES_PER_WORKER,), jnp.int32),
        pltpu.VMEM((CHUNK, D_WORDS), jnp.int32),
        pltpu.VMEM((CHUNK, D_WORDS), jnp.int32),
        pltpu.VMEM((CHUNK, D_WORDS), jnp.int32),
        pltpu.VMEM((CHUNK, D_WORDS), jnp.int32),
        pltpu.VMEM((CHUNK,), jnp.float32),
        pltpu.VMEM((CHUNK,), jnp.float32),
        pltpu.SemaphoreType.DMA,
        pltpu.SemaphoreType.DMA,
        pltpu.SemaphoreType.DMA,
        pltpu.SemaphoreType.DMA,
        pltpu.VMEM_SHARED((N_NODES, D_WORDS), jnp.int32),
    ],
    compiler_params=pltpu.CompilerParams(use_tc_tiling_on_sc=False,
                                         needs_layout_passes=False),
)
def _dot_scores(h_hbm, eidx_hbm, out_hbm,
                su_v, sv_v, hu0, hv0, hu1, hv1, out0, out1,
                sem0, sem1, semw0, semw1, h_sp):
    wid = lax.axis_index("s") * 2 + lax.axis_index("c")
    wbase = wid * EDGES_PER_WORKER
    lane = lax.iota(jnp.int32, 16)
    sel8 = (lane & 8) == 0
    sel4 = (lane & 4) == 0
    sel2 = (lane & 2) == 0
    sel1 = (lane & 1) == 0

    @pl.when(lax.axis_index("s") == 0)
    def _():
        pltpu.sync_copy(h_hbm, h_sp)

    pltpu.sync_copy(eidx_hbm.at[pl.ds(wbase, EDGES_PER_WORKER)], su_v)
    pltpu.sync_copy(eidx_hbm.at[pl.ds(N_EDGES + wbase, EDGES_PER_WORKER)], sv_v)
    plsc.subcore_barrier()

    def start_gather(c, hu, hv, sem):
        off = jnp.minimum(c * CHUNK, LAST_START)
        pltpu.async_copy(h_sp.at[su_v.at[pl.ds(off, CHUNK)]], hu, sem)
        pltpu.async_copy(h_sp.at[sv_v.at[pl.ds(off, CHUNK)]], hv, sem)

    def wait_gather(hu, hv, sem):
        pltpu.make_async_copy(h_hbm.at[pl.ds(0, CHUNK)], hu, sem).wait()
        pltpu.make_async_copy(h_hbm.at[pl.ds(0, CHUNK)], hv, sem).wait()


    def row_partial(hu, hv, r):
        acc = None
        for j in range(D_WORDS // 16):
            wu = hu[r, pl.ds(j * 16, 16)]
            wv = hv[r, pl.ds(j * 16, 16)]
            au = lax.bitcast_convert_type(wu << 16, jnp.float32)
            av = lax.bitcast_convert_type(wv << 16, jnp.float32)
            bu = lax.bitcast_convert_type(wu, jnp.float32)
            bv = lax.bitcast_convert_type(wv, jnp.float32)
            t = au * av + bu * bv
            acc = t if acc is None else acc + t
        return acc

    def compute_chunk(c, hu, hv, outb, semw):
        def group_body(g, carry):
            base_r = g * 16
            m = []
            for a in range(8):
                xa = row_partial(hu, hv, base_r + a)
                xb = row_partial(hu, hv, base_r + a + 8)
                sa = xa + _lane_perm(xa, lane ^ 8)
                sb = xb + _lane_perm(xb, lane ^ 8)
                m.append(jnp.where(sel8, sa, sb))
            u = []
            for a in range(4):
                t1 = m[a] + _lane_perm(m[a], lane ^ 4)
                t2 = m[a + 4] + _lane_perm(m[a + 4], lane ^ 4)
                u.append(jnp.where(sel4, t1, t2))
            w = []
            for a in range(2):
                t1 = u[a] + _lane_perm(u[a], lane ^ 2)
                t2 = u[a + 2] + _lane_perm(u[a + 2], lane ^ 2)
                w.append(jnp.where(sel2, t1, t2))
            t1 = w[0] + _lane_perm(w[0], lane ^ 1)
            t2 = w[1] + _lane_perm(w[1], lane ^ 1)
            outb[pl.ds(base_r, 16)] = jnp.where(sel1, t1, t2)
            return carry

        lax.fori_loop(0, CHUNK // 16, group_body, 0)
        off = jnp.minimum(c * CHUNK, LAST_START)
        pltpu.async_copy(outb, out_hbm.at[pl.ds(wbase + off, CHUNK)], semw)

    def wait_writeback(outb, semw):
        pltpu.make_async_copy(
            outb, out_hbm.at[pl.ds(wbase, CHUNK)], semw).wait()

    bufs = ((hu0, hv0, out0, sem0, semw0),
            (hu1, hv1, out1, sem1, semw1))

    for k in range(RING):
        start_gather(k, bufs[k][0], bufs[k][1], bufs[k][3])

    def round_body(i, carry):
        for k in range(RING):
            c = RING * i + k
            hu, hv, outb, sem, semw = bufs[k]
            wait_gather(hu, hv, sem)

            @pl.when(i > 0)
            def _():
                wait_writeback(outb, semw)

            compute_chunk(c, hu, hv, outb, semw)

            @pl.when(c + RING < NUM_CHUNKS)
            def _():
                start_gather(c + RING, hu, hv, sem)

        return carry

    lax.fori_loop(0, NUM_ROUNDS, round_body, 0)

    for k in range(RING):
        wait_writeback(bufs[k][2], bufs[k][4])


def kernel(h, edge_index):
    eidx = edge_index.astype(jnp.int32).reshape(-1)
    h_packed = lax.bitcast_convert_type(
        h.astype(jnp.bfloat16).reshape(N_NODES, D_WORDS, 2), jnp.int32)
    score = _dot_scores(h_packed, eidx)
    return score.reshape(N_EDGES, 1)


# speedup vs baseline: 12.5169x; 1.0009x over previous
import functools

import jax
import jax.numpy as jnp
from jax import lax
from jax.experimental import pallas as pl
from jax.experimental.pallas import tpu as pltpu
from jax.experimental.pallas import tpu_sc as plsc


def _lane_perm(x, idx):
    dnums = lax.GatherDimensionNumbers(
        offset_dims=(), collapsed_slice_dims=(0,), start_index_map=(0,))
    return lax.gather(x, idx[:, None], dnums, slice_sizes=(1,),
                      mode=lax.GatherScatterMode.PROMISE_IN_BOUNDS)


N_NODES = 10000
N_EDGES = 320000
D_FEAT = 128
D_WORDS = D_FEAT // 2

NUM_WORKERS = 32
EDGES_PER_WORKER = N_EDGES // NUM_WORKERS
CHUNK = 256
RING = 2
NUM_CHUNKS = -(-EDGES_PER_WORKER // CHUNK)
NUM_ROUNDS = NUM_CHUNKS // RING
LAST_START = EDGES_PER_WORKER - CHUNK


@functools.partial(
    pl.kernel,
    mesh=plsc.VectorSubcoreMesh(core_axis_name="c", subcore_axis_name="s"),
    out_type=jax.ShapeDtypeStruct((N_EDGES,), jnp.float32),
    scratch_types=[
        pltpu.VMEM((EDGES_PER_WORKER,), jnp.int32),
        pltpu.VMEM((EDGES_PER_WORKER,), jnp.int32),
        pltpu.VMEM((CHUNK, D_WORDS), jnp.int32),
        pltpu.VMEM((CHUNK, D_WORDS), jnp.int32),
        pltpu.VMEM((CHUNK, D_WORDS), jnp.int32),
        pltpu.VMEM((CHUNK, D_WORDS), jnp.int32),
        pltpu.VMEM((CHUNK,), jnp.float32),
        pltpu.VMEM((CHUNK,), jnp.float32),
        pltpu.SemaphoreType.DMA,
        pltpu.SemaphoreType.DMA,
        pltpu.SemaphoreType.DMA,
        pltpu.SemaphoreType.DMA,
        pltpu.VMEM_SHARED((N_NODES, D_WORDS), jnp.int32),
    ],
    compiler_params=pltpu.CompilerParams(use_tc_tiling_on_sc=False),
)
def _dot_scores(h_hbm, eidx_hbm, out_hbm,
                su_v, sv_v, hu0, hv0, hu1, hv1, out0, out1,
                sem0, sem1, semw0, semw1, h_sp):
    wid = lax.axis_index("s") * 2 + lax.axis_index("c")
    wbase = wid * EDGES_PER_WORKER
    lane = lax.iota(jnp.int32, 16)
    sel8 = (lane & 8) == 0
    sel4 = (lane & 4) == 0
    sel2 = (lane & 2) == 0
    sel1 = (lane & 1) == 0

    @pl.when(lax.axis_index("s") == 0)
    def _():
        pltpu.sync_copy(h_hbm, h_sp)

    pltpu.sync_copy(eidx_hbm.at[pl.ds(wbase, EDGES_PER_WORKER)], su_v)
    pltpu.sync_copy(eidx_hbm.at[pl.ds(N_EDGES + wbase, EDGES_PER_WORKER)], sv_v)
    plsc.subcore_barrier()

    def start_gather(c, hu, hv, sem):
        off = jnp.minimum(c * CHUNK, LAST_START)
        pltpu.async_copy(h_sp.at[su_v.at[pl.ds(off, CHUNK)]], hu, sem)
        pltpu.async_copy(h_sp.at[sv_v.at[pl.ds(off, CHUNK)]], hv, sem)

    def wait_gather(hu, hv, sem):
        pltpu.make_async_copy(h_hbm.at[pl.ds(0, CHUNK)], hu, sem).wait()
        pltpu.make_async_copy(h_hbm.at[pl.ds(0, CHUNK)], hv, sem).wait()


    def row_partial(hu, hv, r):
        acc = None
        for j in range(D_WORDS // 16):
            wu = hu[r, pl.ds(j * 16, 16)]
            wv = hv[r, pl.ds(j * 16, 16)]
            au = lax.bitcast_convert_type(wu << 16, jnp.float32)
            av = lax.bitcast_convert_type(wv << 16, jnp.float32)
            bu = lax.bitcast_convert_type(wu, jnp.float32)
            bv = lax.bitcast_convert_type(wv, jnp.float32)
            t = au * av + bu * bv
            acc = t if acc is None else acc + t
        return acc

    def compute_chunk(c, hu, hv, outb, semw):
        def group_body(g, carry):
            base_r = g * 16
            m = []
            for a in range(8):
                xa = row_partial(hu, hv, base_r + a)
                xb = row_partial(hu, hv, base_r + a + 8)
                sa = xa + _lane_perm(xa, lane ^ 8)
                sb = xb + _lane_perm(xb, lane ^ 8)
                m.append(jnp.where(sel8, sa, sb))
            u = []
            for a in range(4):
                t1 = m[a] + _lane_perm(m[a], lane ^ 4)
                t2 = m[a + 4] + _lane_perm(m[a + 4], lane ^ 4)
                u.append(jnp.where(sel4, t1, t2))
            w = []
            for a in range(2):
                t1 = u[a] + _lane_perm(u[a], lane ^ 2)
                t2 = u[a + 2] + _lane_perm(u[a + 2], lane ^ 2)
                w.append(jnp.where(sel2, t1, t2))
            t1 = w[0] + _lane_perm(w[0], lane ^ 1)
            t2 = w[1] + _lane_perm(w[1], lane ^ 1)
            outb[pl.ds(base_r, 16)] = jnp.where(sel1, t1, t2)
            return carry

        lax.fori_loop(0, CHUNK // 16, group_body, 0)
        off = jnp.minimum(c * CHUNK, LAST_START)
        pltpu.async_copy(outb, out_hbm.at[pl.ds(wbase + off, CHUNK)], semw)

    def wait_writeback(outb, semw):
        pltpu.make_async_copy(
            outb, out_hbm.at[pl.ds(wbase, CHUNK)], semw).wait()

    bufs = ((hu0, hv0, out0, sem0, semw0),
            (hu1, hv1, out1, sem1, semw1))

    for k in range(RING):
        start_gather(k, bufs[k][0], bufs[k][1], bufs[k][3])

    def round_body(i, carry):
        for k in range(RING):
            c = RING * i + k
            hu, hv, outb, sem, semw = bufs[k]
            wait_gather(hu, hv, sem)

            @pl.when(i > 0)
            def _():
                wait_writeback(outb, semw)

            compute_chunk(c, hu, hv, outb, semw)

            @pl.when(c + RING < NUM_CHUNKS)
            def _():
                start_gather(c + RING, hu, hv, sem)

        return carry

    lax.fori_loop(0, NUM_ROUNDS, round_body, 0)

    for k in range(RING):
        wait_writeback(bufs[k][2], bufs[k][4])


def kernel(h, edge_index):
    eidx = edge_index.astype(jnp.int32).reshape(-1)
    h_packed = lax.bitcast_convert_type(
        h.astype(jnp.bfloat16).reshape(N_NODES, D_WORDS, 2), jnp.int32)
    score = _dot_scores(h_packed, eidx)
    return score.reshape(N_EDGES, 1)
